# baseline (speedup 1.0000x reference)
import functools

import jax
import jax.numpy as jnp
from jax import lax
from jax.experimental import pallas as pl
from jax.experimental.pallas import tpu as pltpu
from jax.experimental.pallas import tpu_sc as plsc

H = 8
F = 16
D = H * F
NC = 2
NS = 16
CH = 80


def _proj_body(x_ref, w_ref, asrc_ref, atrg_ref, xp_ref, ss_ref, st_ref):
    xp = jnp.dot(x_ref[...], w_ref[...], preferred_element_type=jnp.float32)
    xp_ref[...] = xp
    ss_ref[...] = jnp.dot(xp, asrc_ref[...], preferred_element_type=jnp.float32)
    st_ref[...] = jnp.dot(xp, atrg_ref[...], preferred_element_type=jnp.float32)


def _project(x_pad, w, a_src, a_trg, np_, blk):
    grid = np_ // blk
    return pl.pallas_call(
        _proj_body,
        grid=(grid,),
        in_specs=[
            pl.BlockSpec((blk, D), lambda i: (i, 0)),
            pl.BlockSpec((D, D), lambda i: (0, 0)),
            pl.BlockSpec((D, F), lambda i: (0, 0)),
            pl.BlockSpec((D, F), lambda i: (0, 0)),
        ],
        out_specs=[
            pl.BlockSpec((blk, D), lambda i: (i, 0)),
            pl.BlockSpec((blk, F), lambda i: (i, 0)),
            pl.BlockSpec((blk, F), lambda i: (i, 0)),
        ],
        out_shape=[
            jax.ShapeDtypeStruct((np_, D), jnp.float32),
            jax.ShapeDtypeStruct((np_, F), jnp.float32),
            jax.ShapeDtypeStruct((np_, F), jnp.float32),
        ],
    )(x_pad, w, a_src, a_trg)


def _sc_body(nch, nh, rows_pt, ss_hbm, st_hbm, xp_hbm, epk_hbm,
             bias_hbm, out_hbm,
             idx0, idx1, idx2, adj0, adj1, ssb0, ssb1, ssb2,
             stb0, stb1, stb2, exb0, exb1, xpb0, xpb1, xpb2, wb0, wb1,
             zb, zb2, bias_v,
             semi0, semi1, semi2, semg0, semg1, semg2, sems0, sems1,
             out_sh, den_sh):
    cid = lax.axis_index("c")
    sid = lax.axis_index("s")
    lo = cid * nh

    idxv = (idx0, idx1, idx2)
    adjv = (adj0, adj1)
    ssb = (ssb0, ssb1, ssb2)
    stb = (stb0, stb1, stb2)
    exb = (exb0, exb1)
    xpb = (xpb0, xpb1, xpb2)
    wb = (wb0, wb1)
    semi = (semi0, semi1, semi2)
    semg = (semg0, semg1, semg2)
    sems = (sems0, sems1)

    pltpu.sync_copy(bias_hbm, bias_v)

    qrows = rows_pt // 8
    def zrow(r, _):
        for k in range(D // 16):
            zb[r, pl.ds(16 * k, 16)] = jnp.zeros((16,), jnp.float32)
        zb2[r, :] = jnp.zeros((16,), jnp.float32)
        return 0
    lax.fori_loop(0, qrows, zrow, 0)
    for q in range(8):
        pltpu.sync_copy(zb, out_sh.at[pl.ds(sid * rows_pt + q * qrows, qrows)])
        pltpu.sync_copy(zb2, den_sh.at[pl.ds(sid * rows_pt + q * qrows, qrows)])
    plsc.subcore_barrier()

    def issue_idx(i, g):
        pltpu.async_copy(epk_hbm.at[sid * nch + i], idxv[g], semi[g])

    def wait_idx(g):
        pltpu.make_async_copy(epk_hbm.at[0], idxv[g], semi[g]).wait()

    def issue_gathers(g):
        pltpu.async_copy(ss_hbm.at[idxv[g].at[0]], ssb[g], semg[g])
        pltpu.async_copy(st_hbm.at[idxv[g].at[1]], stb[g], semg[g])
        pltpu.async_copy(xp_hbm.at[idxv[g].at[0]], xpb[g], semg[g])

    def wait_gathers(g):
        pltpu.make_async_copy(ss_hbm.at[idxv[g].at[0]], ssb[g], semg[g]).wait()
        pltpu.make_async_copy(st_hbm.at[idxv[g].at[1]], stb[g], semg[g]).wait()
        pltpu.make_async_copy(xp_hbm.at[idxv[g].at[0]], xpb[g], semg[g]).wait()

    def issue_scatters(sb):
        pltpu.async_copy(exb[sb], den_sh.at[adjv[sb]], sems[sb], add=True)
        pltpu.async_copy(wb[sb], out_sh.at[adjv[sb]], sems[sb], add=True)

    def wait_scatters(sb):
        pltpu.make_async_copy(exb[sb], den_sh.at[adjv[sb]], sems[sb]).wait()
        pltpu.make_async_copy(wb[sb], out_sh.at[adjv[sb]], sems[sb]).wait()

    def compute(g, sb):
        for v in range(CH // 16):
            rel = idxv[g][1, pl.ds(16 * v, 16)] - lo
            keep = (rel >= 0) & (rel < nh)
            adjv[sb][pl.ds(16 * v, 16)] = jnp.where(keep, rel, nh)

        def edge(e, _):
            sv = ssb[g][e, :] + stb[g][e, :]
            ex = jnp.exp(jnp.maximum(sv, 0.2 * sv))
            exb[sb][e, :] = ex
            for h in range(H):
                sc = ex[h]
                wb[sb][e, pl.ds(16 * h, 16)] = (
                    xpb[g][e, pl.ds(16 * h, 16)] * sc)
            return 0
        lax.fori_loop(0, CH, edge, 0, unroll=2)

    issue_idx(0, 0)
    issue_idx(1, 1)
    issue_idx(2, 2)
    wait_idx(0)
    issue_gathers(0)
    wait_idx(1)
    issue_gathers(1)

    def six(k, _):
        for j in range(6):
            i = 6 * k + j
            g = j % 3
            sb = j % 2
            wait_gathers(g)
            wait_idx((g + 2) % 3)

            @pl.when(i >= 2)
            def _():
                wait_scatters(sb)

            issue_gathers((g + 2) % 3)
            compute(g, sb)
            issue_scatters(sb)
            issue_idx(i + 3, g)
        return 0
    lax.fori_loop(0, nch // 6, six, 0)

    wait_scatters(0)
    wait_scatters(1)
    wait_gathers(0)
    wait_gathers(1)
    wait_idx(2)
    plsc.subcore_barrier()

    def frow(r, _):
        dv = zb2[r, :]
        for h in range(H):
            dh = dv[h] + 1e-16
            val = zb[r, pl.ds(16 * h, 16)] / dh + bias_v[pl.ds(16 * h, 16)]
            zb[r, pl.ds(16 * h, 16)] = jnp.where(
                val > 0, val, jnp.exp(val) - 1.0)
        return 0
    r0 = sid * rows_pt
    for q in range(8):
        pltpu.sync_copy(out_sh.at[pl.ds(r0 + q * qrows, qrows)], zb)
        pltpu.sync_copy(den_sh.at[pl.ds(r0 + q * qrows, qrows)], zb2)
        lax.fori_loop(0, qrows, frow, 0)
        pltpu.sync_copy(zb, out_hbm.at[pl.ds(lo + r0 + q * qrows, qrows)])


def _sc_edge_pass(ss, st, xp, epk, bias, np_, nch):
    nh = np_ // NC
    rows_pt = nh // NS
    mesh = plsc.VectorSubcoreMesh(core_axis_name="c", subcore_axis_name="s")
    fn = pl.kernel(
        functools.partial(_sc_body, nch, nh, rows_pt),
        out_type=jax.ShapeDtypeStruct((np_, D), jnp.float32),
        mesh=mesh,
        compiler_params=pltpu.CompilerParams(use_tc_tiling_on_sc=False),
        scratch_types=(
            [pltpu.VMEM((2, CH), jnp.int32)] * 3
            + [pltpu.VMEM((CH,), jnp.int32)] * 2
            + [pltpu.VMEM((CH, F), jnp.float32)] * 6
            + [pltpu.VMEM((CH, F), jnp.float32)] * 2
            + [pltpu.VMEM((CH, D), jnp.float32)] * 3
            + [pltpu.VMEM((CH, D), jnp.float32)] * 2
            + [
                pltpu.VMEM((nh // NS // 8, D), jnp.float32),
                pltpu.VMEM((nh // NS // 8, F), jnp.float32),
                pltpu.VMEM((D,), jnp.float32),
            ]
            + [pltpu.SemaphoreType.DMA] * 8
            + [
                pltpu.VMEM_SHARED((nh + 8, D), jnp.float32),
                pltpu.VMEM_SHARED((nh + 8, F), jnp.float32),
            ]
        ),
    )
    return fn(ss, st, xp, epk, bias)


def kernel(x, edge_index, W_proj, scoring_src, scoring_trg, bias):
    n, d_in = x.shape
    e = edge_index.shape[1]
    assert d_in == D and W_proj.shape == (d_in, D)

    blk = 512
    np_ = ((n + 1024 - 1) // 1024) * 1024
    nch = -(-e // (NS * CH))
    nch = -(-nch // 6) * 6
    ept = nch * CH
    e_pad = (NS * nch + 3) * CH

    hsel = (jnp.arange(D)[:, None] // F == jnp.arange(F)[None, :])
    a_src = jnp.where(hsel, scoring_src.reshape(-1)[:, None], 0.0).astype(jnp.float32)
    a_trg = jnp.where(hsel, scoring_trg.reshape(-1)[:, None], 0.0).astype(jnp.float32)

    x_pad = jnp.concatenate(
        [x, jnp.zeros((np_ - n, d_in), jnp.float32)], axis=0)
    pad_idx = jnp.full((e_pad - e,), n, jnp.int32)
    src = jnp.concatenate([edge_index[0], pad_idx])
    trg = jnp.concatenate([edge_index[1], pad_idx])
    epk = jnp.stack([src.reshape(-1, CH), trg.reshape(-1, CH)], axis=1)

    xp, ss, st = _project(x_pad, W_proj, a_src, a_trg, np_, blk)
    out = _sc_edge_pass(ss, st, xp, epk,
                        bias.astype(jnp.float32), np_, nch)
    return out[:n]
